# baseline (speedup 1.0000x reference)
import jax
import jax.numpy as jnp
from jax import lax
from jax.experimental import pallas as pl
from jax.experimental.pallas import tpu as pltpu
from jax.experimental.pallas import tpu_sc as plsc

N = 10000
K = 256
D = 128
BK = 64
BN = 2000

NC = 2
NS = 16
NW = NC * NS
CH = 16
NCH = N // CH


def _sub_sc_body(x_hbm, e_hbm, g_hbm, xbuf, ebuf):
    c = lax.axis_index("c")
    s = lax.axis_index("s")
    w = s * NC + c
    nchunks = (NCH - w + NW - 1) // NW

    def chunk_body(i, _):
        r0 = (w + i * NW) * CH
        pltpu.sync_copy(x_hbm.at[pl.ds(r0, CH)], xbuf)
        pltpu.sync_copy(e_hbm.at[pl.ds(r0, CH)], ebuf)

        def row_body(r, _):
            for dc in range(D // 16):
                sl = pl.ds(dc * 16, 16)
                ebuf[r, sl] = ebuf[r, sl] - xbuf[r, sl]
            return 0

        lax.fori_loop(0, CH, row_body, 0)
        pltpu.sync_copy(ebuf, g_hbm.at[pl.ds(r0, CH)])
        return 0

    lax.fori_loop(0, nchunks, chunk_body, 0)


def _tmp_body(tar_ref, x_ref, tmp_ref, xbf_ref):
    @pl.when(pl.program_id(0) == 0)
    def _():
        xbf_ref[...] = x_ref[...].astype(jnp.bfloat16)

    tmp_ref[...] = jnp.dot(
        tar_ref[...].astype(jnp.bfloat16),
        xbf_ref[...],
        preferred_element_type=jnp.float32,
    )


def _out_body(src_a_ref, src_b_ref, tmp_ref, g_ref, out_ref):
    out_ref[...] = (
        jnp.dot(src_a_ref[...], tmp_ref[: K // 2, :], preferred_element_type=jnp.float32)
        + jnp.dot(src_b_ref[...], tmp_ref[K // 2 :, :], preferred_element_type=jnp.float32)
        + g_ref[...]
    )


def kernel(t, x, HG_poi_src, HG_poi_tar, e):
    del t
    mesh = plsc.VectorSubcoreMesh(core_axis_name="c", subcore_axis_name="s")
    g = pl.kernel(
        _sub_sc_body,
        out_type=jax.ShapeDtypeStruct((N, D), jnp.float32),
        mesh=mesh,
        scratch_types=[
            pltpu.VMEM((CH, D), jnp.float32),
            pltpu.VMEM((CH, D), jnp.float32),
        ],
    )(x, e)

    tmp = pl.pallas_call(
        _tmp_body,
        grid=(K // BK,),
        in_specs=[
            pl.BlockSpec((BK, N), lambda i: (i, 0)),
            pl.BlockSpec((N, D), lambda i: (0, 0)),
        ],
        out_specs=pl.BlockSpec((BK, D), lambda i: (i, 0)),
        out_shape=jax.ShapeDtypeStruct((K, D), jnp.float32),
        scratch_shapes=[pltpu.VMEM((N, D), jnp.bfloat16)],
    )(HG_poi_tar, x)

    f = pl.pallas_call(
        _out_body,
        grid=(N // BN,),
        in_specs=[
            pl.BlockSpec((BN, K // 2), lambda i: (i, 0)),
            pl.BlockSpec((BN, K // 2), lambda i: (i, 1)),
            pl.BlockSpec((K, D), lambda i: (0, 0)),
            pl.BlockSpec((BN, D), lambda i: (i, 0)),
        ],
        out_specs=pl.BlockSpec((BN, D), lambda i: (i, 0)),
        out_shape=jax.ShapeDtypeStruct((N, D), jnp.float32),
    )(HG_poi_src, HG_poi_src, tmp, g)
    return f


# speedup vs baseline: 1.4265x; 1.4265x over previous
import jax
import jax.numpy as jnp
from jax import lax
from jax.experimental import pallas as pl
from jax.experimental.pallas import tpu as pltpu
from jax.experimental.pallas import tpu_sc as plsc

N = 10000
K = 256
D = 128
BK = 64
BN = 2000

NC = 2
NS = 16
NW = NC * NS
CH = 128
NCHF = N // CH
CT = N - NCHF * CH


def _sub_sc_body(x_hbm, e_hbm, g_hbm, xbuf, ebuf, semx, seme):
    c = lax.axis_index("c")
    s = lax.axis_index("s")
    w = s * NC + c
    nchunks = (NCHF - w + NW - 1) // NW

    def do_chunk(r0, rows):
        cx = pltpu.async_copy(x_hbm.at[pl.ds(r0, rows)], xbuf.at[pl.ds(0, rows)], semx)
        ce = pltpu.async_copy(e_hbm.at[pl.ds(r0, rows)], ebuf.at[pl.ds(0, rows)], seme)
        cx.wait()
        ce.wait()

        def row_body(r, _):
            for dc in range(D // 16):
                sl = pl.ds(dc * 16, 16)
                ebuf[r, sl] = ebuf[r, sl] - xbuf[r, sl]
            return 0

        lax.fori_loop(0, rows, row_body, 0)
        pltpu.sync_copy(ebuf.at[pl.ds(0, rows)], g_hbm.at[pl.ds(r0, rows)])

    def chunk_body(i, _):
        do_chunk((w + i * NW) * CH, CH)
        return 0

    lax.fori_loop(0, nchunks, chunk_body, 0)

    @pl.when(w == NW - 1)
    def _tail():
        do_chunk(NCHF * CH, CT)


def _tmp_body(tar_ref, x_ref, tmp_ref, xbf_ref):
    @pl.when(pl.program_id(0) == 0)
    def _():
        xbf_ref[...] = x_ref[...].astype(jnp.bfloat16)

    tmp_ref[...] = jnp.dot(
        tar_ref[...].astype(jnp.bfloat16),
        xbf_ref[...],
        preferred_element_type=jnp.float32,
    )


def _out_body(src_a_ref, src_b_ref, tmp_ref, g_ref, out_ref):
    out_ref[...] = (
        jnp.dot(src_a_ref[...], tmp_ref[: K // 2, :], preferred_element_type=jnp.float32)
        + jnp.dot(src_b_ref[...], tmp_ref[K // 2 :, :], preferred_element_type=jnp.float32)
        + g_ref[...]
    )


def kernel(t, x, HG_poi_src, HG_poi_tar, e):
    del t
    mesh = plsc.VectorSubcoreMesh(core_axis_name="c", subcore_axis_name="s")
    g = pl.kernel(
        _sub_sc_body,
        out_type=jax.ShapeDtypeStruct((N, D), jnp.float32),
        mesh=mesh,
        scratch_types=[
            pltpu.VMEM((CH, D), jnp.float32),
            pltpu.VMEM((CH, D), jnp.float32),
            pltpu.SemaphoreType.DMA,
            pltpu.SemaphoreType.DMA,
        ],
    )(x, e)

    tmp = pl.pallas_call(
        _tmp_body,
        grid=(K // BK,),
        in_specs=[
            pl.BlockSpec((BK, N), lambda i: (i, 0)),
            pl.BlockSpec((N, D), lambda i: (0, 0)),
        ],
        out_specs=pl.BlockSpec((BK, D), lambda i: (i, 0)),
        out_shape=jax.ShapeDtypeStruct((K, D), jnp.float32),
        scratch_shapes=[pltpu.VMEM((N, D), jnp.bfloat16)],
    )(HG_poi_tar, x)

    f = pl.pallas_call(
        _out_body,
        grid=(N // BN,),
        in_specs=[
            pl.BlockSpec((BN, K // 2), lambda i: (i, 0)),
            pl.BlockSpec((BN, K // 2), lambda i: (i, 1)),
            pl.BlockSpec((K, D), lambda i: (0, 0)),
            pl.BlockSpec((BN, D), lambda i: (i, 0)),
        ],
        out_specs=pl.BlockSpec((BN, D), lambda i: (i, 0)),
        out_shape=jax.ShapeDtypeStruct((N, D), jnp.float32),
    )(HG_poi_src, HG_poi_src, tmp, g)
    return f


# speedup vs baseline: 2.4604x; 1.7248x over previous
import jax
import jax.numpy as jnp
from jax.experimental import pallas as pl
from jax.experimental.pallas import tpu as pltpu

N = 10000
K = 256
D = 128
BK = 128
BN = 2000
P1 = K // BK


def _fused_body(tar_ref, src_a_ref, src_b_ref, x_ref, e_ref, out_ref, tmp_ref, xbf_ref):
    i = pl.program_id(0)

    @pl.when(i == 0)
    def _():
        xbf_ref[...] = x_ref[...].astype(jnp.bfloat16)

    @pl.when(i < P1)
    def _():
        tmp_ref[pl.ds(i * BK, BK), :] = jnp.dot(
            tar_ref[...].astype(jnp.bfloat16),
            xbf_ref[...],
            preferred_element_type=jnp.float32,
        )

    @pl.when(i >= P1)
    def _():
        j = i - P1
        out_ref[...] = (
            jnp.dot(src_a_ref[...], tmp_ref[: K // 2, :], preferred_element_type=jnp.float32)
            + jnp.dot(src_b_ref[...], tmp_ref[K // 2 :, :], preferred_element_type=jnp.float32)
            + e_ref[...]
            - x_ref[pl.ds(j * BN, BN), :]
        )


def kernel(t, x, HG_poi_src, HG_poi_tar, e):
    del t
    f = pl.pallas_call(
        _fused_body,
        grid=(P1 + N // BN,),
        in_specs=[
            pl.BlockSpec((BK, N), lambda i: (jnp.minimum(i, P1 - 1), 0)),
            pl.BlockSpec((BN, K // 2), lambda i: (jnp.maximum(i - P1, 0), 0)),
            pl.BlockSpec((BN, K // 2), lambda i: (jnp.maximum(i - P1, 0), 1)),
            pl.BlockSpec((N, D), lambda i: (0, 0)),
            pl.BlockSpec((BN, D), lambda i: (jnp.maximum(i - P1, 0), 0)),
        ],
        out_specs=pl.BlockSpec((BN, D), lambda i: (jnp.maximum(i - P1, 0), 0)),
        out_shape=jax.ShapeDtypeStruct((N, D), jnp.float32),
        scratch_shapes=[
            pltpu.VMEM((K, D), jnp.float32),
            pltpu.VMEM((N, D), jnp.bfloat16),
        ],
    )(HG_poi_tar, HG_poi_src, HG_poi_src, x, e)
    return f


# speedup vs baseline: 2.4887x; 1.0115x over previous
import jax
import jax.numpy as jnp
from jax.experimental import pallas as pl
from jax.experimental.pallas import tpu as pltpu

N = 10000
K = 256
D = 128
BK = 128
BN = 2000
P1 = K // BK
NH = N // 2


def _fused_body(
    tar_ref, src_hbm, x_ref, e_ref, out_ref, tmp_ref, xbf_ref, src_ref, sem_a, sem_b
):
    i = pl.program_id(0)

    @pl.when(i == 0)
    def _():
        pltpu.make_async_copy(
            src_hbm.at[pl.ds(0, NH)], src_ref.at[pl.ds(0, NH)], sem_a
        ).start()
        pltpu.make_async_copy(
            src_hbm.at[pl.ds(NH, NH)], src_ref.at[pl.ds(NH, NH)], sem_b
        ).start()
        xbf_ref[...] = x_ref[...].astype(jnp.bfloat16)

    @pl.when(i < P1)
    def _():
        tmp_ref[pl.ds(i * BK, BK), :] = jnp.dot(
            tar_ref[...].astype(jnp.bfloat16),
            xbf_ref[...],
            preferred_element_type=jnp.float32,
        )

    @pl.when(i == P1)
    def _():
        pltpu.make_async_copy(
            src_hbm.at[pl.ds(0, NH)], src_ref.at[pl.ds(0, NH)], sem_a
        ).wait()
        pltpu.make_async_copy(
            src_hbm.at[pl.ds(NH, NH)], src_ref.at[pl.ds(NH, NH)], sem_b
        ).wait()

    @pl.when(i >= P1)
    def _():
        j = i - P1
        src_blk = src_ref[pl.ds(j * BN, BN), :]
        out_ref[...] = (
            jnp.dot(src_blk[:, : K // 2], tmp_ref[: K // 2, :], preferred_element_type=jnp.float32)
            + jnp.dot(src_blk[:, K // 2 :], tmp_ref[K // 2 :, :], preferred_element_type=jnp.float32)
            + e_ref[...]
            - x_ref[pl.ds(j * BN, BN), :]
        )


def kernel(t, x, HG_poi_src, HG_poi_tar, e):
    del t
    f = pl.pallas_call(
        _fused_body,
        grid=(P1 + N // BN,),
        in_specs=[
            pl.BlockSpec((BK, N), lambda i: (jnp.minimum(i, P1 - 1), 0)),
            pl.BlockSpec(memory_space=pl.ANY),
            pl.BlockSpec((N, D), lambda i: (0, 0)),
            pl.BlockSpec((BN, D), lambda i: (jnp.maximum(i - P1, 0), 0)),
        ],
        out_specs=pl.BlockSpec((BN, D), lambda i: (jnp.maximum(i - P1, 0), 0)),
        out_shape=jax.ShapeDtypeStruct((N, D), jnp.float32),
        scratch_shapes=[
            pltpu.VMEM((K, D), jnp.float32),
            pltpu.VMEM((N, D), jnp.bfloat16),
            pltpu.VMEM((N, K), jnp.float32),
            pltpu.SemaphoreType.DMA,
            pltpu.SemaphoreType.DMA,
        ],
    )(HG_poi_tar, HG_poi_src, x, e)
    return f
